# baseline (speedup 1.0000x reference)
import functools

import jax
import jax.numpy as jnp
from jax import lax
from jax.experimental import pallas as pl
from jax.experimental.pallas import tpu as pltpu
from jax.experimental.pallas import tpu_sc as plsc

_NUM_WORKERS = 32
_R = 4


def _emb_kernel(x_hbm, table_hbm, out_hbm, idx_v, rows_v, gsem0, gsem1,
                isem, ssem0, ssem1, *, steps, seq_len, d):
    wid = lax.axis_index("s") * 2 + lax.axis_index("c")
    row_base = wid * (steps * _R)
    blk = _R * seq_len
    gsems = (gsem0, gsem1)
    ssems = (ssem0, ssem1)
    splits = []
    off = 0
    while off < seq_len:
        g = min(128, seq_len - off)
        splits.append((off, g))
        off += g

    def issue_gathers(b):
        for r in range(_R):
            for (o, g) in splits:
                pltpu.async_copy(
                    table_hbm.at[idx_v.at[b, r, pl.ds(o, g)]],
                    rows_v.at[b].at[pl.ds(r * seq_len + o, g)],
                    gsems[b],
                )

    def drain_gathers(b):
        pltpu.make_async_copy(
            out_hbm.at[pl.ds(0, blk)], rows_v.at[b], gsems[b]
        ).wait()

    pltpu.sync_copy(x_hbm.at[pl.ds(row_base, _R)], idx_v.at[0])

    @pl.loop(0, steps, step=2)
    def pair(i0):
        for b in range(2):
            i = i0 + b
            brow = row_base + i * _R
            nb = 1 - b

            @pl.when(i >= 1)
            def _():
                pltpu.make_async_copy(
                    x_hbm.at[pl.ds(0, _R)], idx_v.at[b], isem
                ).wait()

            @pl.when(i >= 2)
            def _():
                pltpu.make_async_copy(
                    rows_v.at[b], out_hbm.at[pl.ds(0, blk)], ssems[b]
                ).wait()

            issue_gathers(b)

            @pl.when(i >= 1)
            def _():
                drain_gathers(nb)
                pltpu.async_copy(
                    rows_v.at[nb],
                    out_hbm.at[pl.ds((brow - _R) * seq_len, blk)],
                    ssems[nb],
                )

            @pl.when(i + 1 < steps)
            def _():
                pltpu.async_copy(
                    x_hbm.at[pl.ds(brow + _R, _R)], idx_v.at[nb], isem
                )

    drain_gathers(1)
    pltpu.async_copy(
        rows_v.at[1],
        out_hbm.at[pl.ds((row_base + (steps - 1) * _R) * seq_len, blk)],
        ssems[1],
    )
    for b in range(2):
        pltpu.make_async_copy(
            rows_v.at[b], out_hbm.at[pl.ds(0, blk)], ssems[b]
        ).wait()


def kernel(x, table):
    bsz, l = x.shape
    v, d = table.shape
    assert bsz % (_NUM_WORKERS * _R) == 0
    steps = bsz // (_NUM_WORKERS * _R)
    assert steps % 2 == 0
    blk = _R * l

    mesh = plsc.VectorSubcoreMesh(core_axis_name="c", subcore_axis_name="s")
    emb = pl.kernel(
        functools.partial(_emb_kernel, steps=steps, seq_len=l, d=d),
        out_type=jax.ShapeDtypeStruct((bsz * l, d), jnp.float32),
        mesh=mesh,
        scratch_types=[
            pltpu.VMEM((2, _R, l), jnp.int32),
            pltpu.VMEM((2, blk, d), jnp.float32),
            pltpu.SemaphoreType.DMA,
            pltpu.SemaphoreType.DMA,
            pltpu.SemaphoreType.DMA,
            pltpu.SemaphoreType.DMA,
            pltpu.SemaphoreType.DMA,
        ],
        compiler_params=pltpu.CompilerParams(use_tc_tiling_on_sc=False),
    )
    stride = 128 // d
    tpad = jnp.pad(table, ((0, 0), (0, 128 - d))).reshape(v * stride, d)
    return emb((x * stride).astype(jnp.int32), tpad).reshape(bsz, l * d)


# speedup vs baseline: 1.0013x; 1.0013x over previous
import functools

import jax
import jax.numpy as jnp
from jax import lax
from jax.experimental import pallas as pl
from jax.experimental.pallas import tpu as pltpu
from jax.experimental.pallas import tpu_sc as plsc

_NUM_WORKERS = 32
_R = 8


def _emb_kernel(x_hbm, table_hbm, out_hbm, idx_v, rows_v, gsem0, gsem1,
                isem, ssem0, ssem1, *, steps, seq_len, d):
    wid = lax.axis_index("s") * 2 + lax.axis_index("c")
    row_base = wid * (steps * _R)
    blk = _R * seq_len
    gsems = (gsem0, gsem1)
    ssems = (ssem0, ssem1)
    splits = []
    off = 0
    while off < seq_len:
        g = min(128, seq_len - off)
        splits.append((off, g))
        off += g

    def issue_gathers(b):
        for r in range(_R):
            for (o, g) in splits:
                pltpu.async_copy(
                    table_hbm.at[idx_v.at[b, r, pl.ds(o, g)]],
                    rows_v.at[b].at[pl.ds(r * seq_len + o, g)],
                    gsems[b],
                )

    def drain_gathers(b):
        pltpu.make_async_copy(
            out_hbm.at[pl.ds(0, blk)], rows_v.at[b], gsems[b]
        ).wait()

    pltpu.sync_copy(x_hbm.at[pl.ds(row_base, _R)], idx_v.at[0])

    @pl.loop(0, steps, step=2)
    def pair(i0):
        for b in range(2):
            i = i0 + b
            brow = row_base + i * _R
            nb = 1 - b

            @pl.when(i >= 1)
            def _():
                pltpu.make_async_copy(
                    x_hbm.at[pl.ds(0, _R)], idx_v.at[b], isem
                ).wait()

            @pl.when(i >= 2)
            def _():
                pltpu.make_async_copy(
                    rows_v.at[b], out_hbm.at[pl.ds(0, blk)], ssems[b]
                ).wait()

            issue_gathers(b)

            @pl.when(i >= 1)
            def _():
                drain_gathers(nb)
                pltpu.async_copy(
                    rows_v.at[nb],
                    out_hbm.at[pl.ds((brow - _R) * seq_len, blk)],
                    ssems[nb],
                )

            @pl.when(i + 1 < steps)
            def _():
                pltpu.async_copy(
                    x_hbm.at[pl.ds(brow + _R, _R)], idx_v.at[nb], isem
                )

    drain_gathers(1)
    pltpu.async_copy(
        rows_v.at[1],
        out_hbm.at[pl.ds((row_base + (steps - 1) * _R) * seq_len, blk)],
        ssems[1],
    )
    for b in range(2):
        pltpu.make_async_copy(
            rows_v.at[b], out_hbm.at[pl.ds(0, blk)], ssems[b]
        ).wait()


def kernel(x, table):
    bsz, l = x.shape
    v, d = table.shape
    assert bsz % (_NUM_WORKERS * _R) == 0
    steps = bsz // (_NUM_WORKERS * _R)
    assert steps % 2 == 0
    blk = _R * l

    mesh = plsc.VectorSubcoreMesh(core_axis_name="c", subcore_axis_name="s")
    emb = pl.kernel(
        functools.partial(_emb_kernel, steps=steps, seq_len=l, d=d),
        out_type=jax.ShapeDtypeStruct((bsz * l, d), jnp.float32),
        mesh=mesh,
        scratch_types=[
            pltpu.VMEM((2, _R, l), jnp.int32),
            pltpu.VMEM((2, blk, d), jnp.float32),
            pltpu.SemaphoreType.DMA,
            pltpu.SemaphoreType.DMA,
            pltpu.SemaphoreType.DMA,
            pltpu.SemaphoreType.DMA,
            pltpu.SemaphoreType.DMA,
        ],
        compiler_params=pltpu.CompilerParams(use_tc_tiling_on_sc=False),
    )
    stride = 128 // d
    tpad = jnp.pad(table, ((0, 0), (0, 128 - d))).reshape(v * stride, d)
    return emb((x * stride).astype(jnp.int32), tpad).reshape(bsz, l * d)


# speedup vs baseline: 1.0028x; 1.0015x over previous
import functools

import jax
import jax.numpy as jnp
from jax import lax
from jax.experimental import pallas as pl
from jax.experimental.pallas import tpu as pltpu
from jax.experimental.pallas import tpu_sc as plsc

_NUM_WORKERS = 32
_R = 8


def _emb_kernel(x_hbm, table_hbm, out_hbm, idx_v, rows_v, gsem0, gsem1,
                isem, ssem0, ssem1, *, steps, seq_len, d):
    wid = lax.axis_index("s") * 2 + lax.axis_index("c")
    row_base = wid * (steps * _R)
    blk = _R * seq_len
    gsems = (gsem0, gsem1)
    ssems = (ssem0, ssem1)
    splits = []
    off = 0
    while off < seq_len:
        g = min(128, seq_len - off)
        splits.append((off, g))
        off += g

    def issue_gathers(b):
        for r in range(_R):
            for (o, g) in splits:
                pltpu.async_copy(
                    table_hbm.at[idx_v.at[b, r, pl.ds(o, g)]],
                    rows_v.at[b].at[pl.ds(r * seq_len + o, g)],
                    gsems[b],
                )

    def drain_gathers(b):
        pltpu.make_async_copy(
            out_hbm.at[pl.ds(0, blk)], rows_v.at[b], gsems[b]
        ).wait()

    pltpu.sync_copy(x_hbm.at[pl.ds(row_base, _R)], idx_v.at[0])

    @pl.loop(0, steps, step=2)
    def pair(i0):
        for b in range(2):
            i = i0 + b
            brow = row_base + i * _R
            nb = 1 - b

            @pl.when(i >= 1)
            def _():
                pltpu.make_async_copy(
                    x_hbm.at[pl.ds(0, _R)], idx_v.at[b], isem
                ).wait()

            @pl.when(i >= 2)
            def _():
                pltpu.make_async_copy(
                    rows_v.at[b], out_hbm.at[pl.ds(0, blk)], ssems[b]
                ).wait()

            issue_gathers(b)

            @pl.when(i >= 1)
            def _():
                drain_gathers(nb)
                pltpu.async_copy(
                    rows_v.at[nb],
                    out_hbm.at[pl.ds((brow - _R) * seq_len, blk)],
                    ssems[nb],
                )

            @pl.when(i + 1 < steps)
            def _():
                pltpu.async_copy(
                    x_hbm.at[pl.ds(brow + _R, _R)], idx_v.at[nb], isem
                )

    drain_gathers(1)
    pltpu.async_copy(
        rows_v.at[1],
        out_hbm.at[pl.ds((row_base + (steps - 1) * _R) * seq_len, blk)],
        ssems[1],
    )
    for b in range(2):
        pltpu.make_async_copy(
            rows_v.at[b], out_hbm.at[pl.ds(0, blk)], ssems[b]
        ).wait()


def kernel(x, table):
    bsz, l = x.shape
    v, d = table.shape
    assert bsz % (_NUM_WORKERS * _R) == 0
    steps = bsz // (_NUM_WORKERS * _R)
    assert steps % 2 == 0
    blk = _R * l

    mesh = plsc.VectorSubcoreMesh(core_axis_name="c", subcore_axis_name="s")
    emb = pl.kernel(
        functools.partial(_emb_kernel, steps=steps, seq_len=l, d=d),
        out_type=jax.ShapeDtypeStruct((bsz * l, d), jnp.float32),
        mesh=mesh,
        scratch_types=[
            pltpu.VMEM((2, _R, l), jnp.int32),
            pltpu.VMEM((2, blk, d), jnp.float32),
            pltpu.SemaphoreType.DMA,
            pltpu.SemaphoreType.DMA,
            pltpu.SemaphoreType.DMA,
            pltpu.SemaphoreType.DMA,
            pltpu.SemaphoreType.DMA,
        ],
        compiler_params=pltpu.CompilerParams(use_tc_tiling_on_sc=False),
    )
    assert 128 % d == 0
    stride = 128 // d
    tpad = jnp.pad(table, ((0, 0), (0, 128 - d))).reshape(v * stride, d)
    return emb((x * stride).astype(jnp.int32), tpad).reshape(bsz, l * d)
